# baseline (speedup 1.0000x reference)
import functools

import numpy as np
import jax
import jax.numpy as jnp
from jax import lax
from jax.experimental import pallas as pl
from jax.experimental.pallas import tpu as pltpu
from jax.experimental.pallas import tpu_sc as plsc

IN_SIZE = 65536
OUT_SIZE = 65536
CONN = 16
BATCH = 64

NC = 2
NS = 16
NW = NC * NS
ROWS_PER_W = OUT_SIZE // NW
UNIT = 32
NUNIT = ROWS_PER_W // UNIT

_mesh = plsc.VectorSubcoreMesh(core_axis_name="c", subcore_axis_name="s")


@functools.partial(
    pl.kernel,
    mesh=_mesh,
    out_type=jax.ShapeDtypeStruct((OUT_SIZE, BATCH), jnp.float32),
    scratch_types=[
        pltpu.VMEM((CONN, ROWS_PER_W), jnp.int32),
        pltpu.VMEM((CONN, ROWS_PER_W), jnp.float32),
        pltpu.VMEM((2, CONN * UNIT, BATCH // 2), jnp.int32),
        pltpu.VMEM((2, UNIT, BATCH), jnp.float32),
        pltpu.SemaphoreType.DMA,
        pltpu.SemaphoreType.DMA,
        pltpu.SemaphoreType.DMA,
        pltpu.SemaphoreType.DMA,
    ],
    compiler_params=pltpu.CompilerParams(use_tc_tiling_on_sc=False,
                                         needs_layout_passes=False),
)
def _spmm_sc(xt_hbm, idx_hbm, val_hbm, out_hbm, idx_v, val_v, rows_v, out_v,
             sem_g0, sem_g1, sem_o0, sem_o1):
    wid = lax.axis_index("s") * NC + lax.axis_index("c")
    row_base = wid * ROWS_PER_W
    sems_g = (sem_g0, sem_g1)
    sems_o = (sem_o0, sem_o1)

    pltpu.sync_copy(idx_hbm.at[:, pl.ds(row_base, ROWS_PER_W)], idx_v)
    pltpu.sync_copy(val_hbm.at[:, pl.ds(row_base, ROWS_PER_W)], val_v)

    kiota = lax.iota(jnp.int32, 16)
    iota2 = kiota * 2
    himask = jnp.full((16,), -65536, jnp.int32)

    def fire(u, buf):
        for h in range(CONN):
            pltpu.async_copy(xt_hbm.at[idx_v.at[h, pl.ds(u * UNIT, UNIT)]],
                             rows_v.at[buf, pl.ds(h * UNIT, UNIT)],
                             sems_g[buf])

    def drain_g(buf):
        for h in range(CONN):
            pltpu.make_async_copy(xt_hbm.at[idx_v.at[0, pl.ds(0, UNIT)]],
                                  rows_v.at[buf, pl.ds(h * UNIT, UNIT)],
                                  sems_g[buf]).wait()

    def compute(u, buf):
        def col(cc, carry):
            vv = plsc.load_gather(val_v, [kiota,
                                          jnp.full((16,), u * UNIT + cc,
                                                   jnp.int32)])
            accs = [jnp.zeros((16,), jnp.float32) for _ in range(4)]
            for k in range(CONN):
                s = vv[k]
                for half in range(2):
                    bits = rows_v[buf, k * UNIT + cc, pl.ds(half * 16, 16)]
                    even = plsc.bitcast(lax.shift_left(bits, 16), jnp.float32)
                    odd = plsc.bitcast(bits & himask, jnp.float32)
                    accs[half * 2] = accs[half * 2] + s * even
                    accs[half * 2 + 1] = accs[half * 2 + 1] + s * odd
            ccv = jnp.full((16,), cc, jnp.int32)
            for j in range(4):
                boff = iota2 + ((j // 2) * 32 + (j % 2))
                plsc.store_scatter(out_v.at[buf], [ccv, boff], accs[j])
            return carry

        lax.fori_loop(0, UNIT, col, 0, unroll=2)

    def write_out(u, buf):
        pltpu.async_copy(out_v.at[buf],
                         out_hbm.at[pl.ds(row_base + u * UNIT, UNIT)],
                         sems_o[buf])

    def wait_out(buf):
        pltpu.make_async_copy(out_v.at[buf],
                              out_hbm.at[pl.ds(row_base, UNIT)],
                              sems_o[buf]).wait()

    fire(0, 0)
    fire(1, 1)
    drain_g(0)
    compute(0, 0)
    fire(2, 0)
    write_out(0, 0)
    drain_g(1)
    compute(1, 1)
    fire(3, 1)
    write_out(1, 1)

    def pair(t, carry):
        u0 = t * 2
        drain_g(0)
        wait_out(0)
        compute(u0, 0)
        fire(u0 + 2, 0)
        write_out(u0, 0)
        drain_g(1)
        wait_out(1)
        compute(u0 + 1, 1)
        fire(u0 + 3, 1)
        write_out(u0 + 1, 1)
        return carry

    lax.fori_loop(1, NUNIT // 2 - 1, pair, 0)

    drain_g(0)
    wait_out(0)
    compute(NUNIT - 2, 0)
    write_out(NUNIT - 2, 0)
    drain_g(1)
    wait_out(1)
    compute(NUNIT - 1, 1)
    write_out(NUNIT - 1, 1)
    wait_out(0)
    wait_out(1)


def kernel(x, indices, values):
    xb = lax.bitcast_convert_type(x, jnp.uint32)
    rnd = (xb + jnp.uint32(0x7FFF) + ((xb >> 16) & jnp.uint32(1))) >> 16
    words = rnd[0::2, :] | (rnd[1::2, :] << 16)
    xt = lax.bitcast_convert_type(words.T, jnp.int32)
    idx2d = indices[1].reshape(CONN, OUT_SIZE)
    val2d = values.reshape(CONN, OUT_SIZE)
    outt = _spmm_sc(xt, idx2d, val2d)
    return outt.T


# speedup vs baseline: 3.2033x; 3.2033x over previous
import functools

import numpy as np
import jax
import jax.numpy as jnp
from jax import lax
from jax.experimental import pallas as pl
from jax.experimental.pallas import tpu as pltpu
from jax.experimental.pallas import tpu_sc as plsc

IN_SIZE = 65536
OUT_SIZE = 65536
CONN = 16
BATCH = 64

NC = 2
NS = 16
NW = NC * NS
ROWS_PER_W = OUT_SIZE // NW
UNIT = 32
NUNIT = ROWS_PER_W // UNIT

_mesh = plsc.VectorSubcoreMesh(core_axis_name="c", subcore_axis_name="s")


@functools.partial(
    pl.kernel,
    mesh=_mesh,
    out_type=jax.ShapeDtypeStruct((OUT_SIZE, BATCH), jnp.float32),
    scratch_types=[
        pltpu.VMEM((CONN, ROWS_PER_W), jnp.int32),
        pltpu.VMEM((CONN, ROWS_PER_W), jnp.float32),
        pltpu.VMEM((2, CONN * UNIT, BATCH), jnp.bfloat16),
        pltpu.VMEM((2, UNIT, BATCH), jnp.float32),
        pltpu.SemaphoreType.DMA,
        pltpu.SemaphoreType.DMA,
        pltpu.SemaphoreType.DMA,
        pltpu.SemaphoreType.DMA,
    ],
    compiler_params=pltpu.CompilerParams(use_tc_tiling_on_sc=False,
                                         needs_layout_passes=False),
)
def _spmm_sc(xt_hbm, idx_hbm, val_hbm, out_hbm, idx_v, val_v, rows_v, out_v,
             sem_g0, sem_g1, sem_o0, sem_o1):
    wid = lax.axis_index("s") * NC + lax.axis_index("c")
    row_base = wid * ROWS_PER_W
    sems_g = (sem_g0, sem_g1)
    sems_o = (sem_o0, sem_o1)

    for k in range(CONN):
        pltpu.sync_copy(idx_hbm.at[pl.ds(k * OUT_SIZE + row_base, ROWS_PER_W)],
                        idx_v.at[k])
        pltpu.sync_copy(val_hbm.at[pl.ds(k * OUT_SIZE + row_base, ROWS_PER_W)],
                        val_v.at[k])

    kiota = lax.iota(jnp.int32, 16)
    iota2 = kiota * 2
    himask = jnp.full((16,), -65536, jnp.int32)

    def fire(u, buf):
        for h in range(CONN):
            pltpu.async_copy(xt_hbm.at[idx_v.at[h, pl.ds(u * UNIT, UNIT)]],
                             rows_v.at[buf, pl.ds(h * UNIT, UNIT)],
                             sems_g[buf])

    def drain_g(buf):
        for h in range(CONN):
            pltpu.make_async_copy(xt_hbm.at[idx_v.at[0, pl.ds(0, UNIT)]],
                                  rows_v.at[buf, pl.ds(h * UNIT, UNIT)],
                                  sems_g[buf]).wait()

    def compute(u, buf):
        def col(cc, carry):
            vv = plsc.load_gather(val_v, [kiota,
                                          jnp.full((16,), u * UNIT + cc,
                                                   jnp.int32)])
            accs = [jnp.zeros((16,), jnp.float32) for _ in range(4)]
            for k in range(CONN):
                s = vv[k]
                for half in range(2):
                    raw = rows_v[buf, k * UNIT + cc, pl.ds(half * 32, 32)]
                    bits = plsc.bitcast(raw, jnp.int32)
                    even = plsc.bitcast(lax.shift_left(bits, 16), jnp.float32)
                    odd = plsc.bitcast(bits & himask, jnp.float32)
                    accs[half * 2] = accs[half * 2] + s * even
                    accs[half * 2 + 1] = accs[half * 2 + 1] + s * odd
            ccv = jnp.full((16,), cc, jnp.int32)
            for j in range(4):
                boff = iota2 + ((j // 2) * 32 + (j % 2))
                plsc.store_scatter(out_v.at[buf], [ccv, boff], accs[j])
            return carry

        lax.fori_loop(0, UNIT, col, 0, unroll=4)

    def write_out(u, buf):
        pltpu.async_copy(out_v.at[buf],
                         out_hbm.at[pl.ds(row_base + u * UNIT, UNIT)],
                         sems_o[buf])

    def wait_out(buf):
        pltpu.make_async_copy(out_v.at[buf],
                              out_hbm.at[pl.ds(row_base, UNIT)],
                              sems_o[buf]).wait()

    fire(0, 0)
    fire(1, 1)
    drain_g(0)
    compute(0, 0)
    fire(2, 0)
    write_out(0, 0)
    drain_g(1)
    compute(1, 1)
    fire(3, 1)
    write_out(1, 1)

    def pair(t, carry):
        u0 = t * 2
        drain_g(0)
        wait_out(0)
        compute(u0, 0)
        fire(u0 + 2, 0)
        write_out(u0, 0)
        drain_g(1)
        wait_out(1)
        compute(u0 + 1, 1)
        fire(u0 + 3, 1)
        write_out(u0 + 1, 1)
        return carry

    lax.fori_loop(1, NUNIT // 2 - 1, pair, 0)

    drain_g(0)
    wait_out(0)
    compute(NUNIT - 2, 0)
    write_out(NUNIT - 2, 0)
    drain_g(1)
    wait_out(1)
    compute(NUNIT - 1, 1)
    write_out(NUNIT - 1, 1)
    wait_out(0)
    wait_out(1)


def kernel(x, indices, values):
    xt = x.astype(jnp.bfloat16).T
    outt = _spmm_sc(xt, indices[1], values)
    return outt.T


# speedup vs baseline: 3.4302x; 1.0708x over previous
import functools

import numpy as np
import jax
import jax.numpy as jnp
from jax import lax
from jax.experimental import pallas as pl
from jax.experimental.pallas import tpu as pltpu
from jax.experimental.pallas import tpu_sc as plsc

IN_SIZE = 65536
OUT_SIZE = 65536
CONN = 16
BATCH = 64

NC = 2
NS = 16
NW = NC * NS
ROWS_PER_W = OUT_SIZE // NW
UNIT = 32
NUNIT = ROWS_PER_W // UNIT

_mesh = plsc.VectorSubcoreMesh(core_axis_name="c", subcore_axis_name="s")


@functools.partial(
    pl.kernel,
    mesh=_mesh,
    out_type=jax.ShapeDtypeStruct((OUT_SIZE, BATCH), jnp.float32),
    scratch_types=[
        pltpu.VMEM((CONN, ROWS_PER_W), jnp.int32),
        pltpu.VMEM((CONN, ROWS_PER_W), jnp.float32),
        pltpu.VMEM((2, CONN * UNIT, BATCH), jnp.bfloat16),
        pltpu.VMEM((2, UNIT, BATCH), jnp.float32),
        pltpu.SemaphoreType.DMA,
        pltpu.SemaphoreType.DMA,
        pltpu.SemaphoreType.DMA,
        pltpu.SemaphoreType.DMA,
    ],
    compiler_params=pltpu.CompilerParams(use_tc_tiling_on_sc=False,
                                         needs_layout_passes=False),
)
def _spmm_sc(xt_hbm, idx_hbm, val_hbm, out_hbm, idx_v, val_v, rows_v, out_v,
             sem_g0, sem_g1, sem_o0, sem_o1):
    wid = lax.axis_index("s") * NC + lax.axis_index("c")
    row_base = wid * ROWS_PER_W
    sems_g = (sem_g0, sem_g1)
    sems_o = (sem_o0, sem_o1)

    for k in range(CONN):
        pltpu.async_copy(idx_hbm.at[pl.ds(k * OUT_SIZE + row_base, ROWS_PER_W)],
                         idx_v.at[k], sem_g0)
        pltpu.async_copy(val_hbm.at[pl.ds(k * OUT_SIZE + row_base, ROWS_PER_W)],
                         val_v.at[k], sem_g1)
    for k in range(CONN):
        pltpu.make_async_copy(idx_hbm.at[pl.ds(row_base, ROWS_PER_W)],
                              idx_v.at[k], sem_g0).wait()
        pltpu.make_async_copy(val_hbm.at[pl.ds(row_base, ROWS_PER_W)],
                              val_v.at[k], sem_g1).wait()

    kiota = lax.iota(jnp.int32, 16)
    iota2 = kiota * 2
    himask = jnp.full((16,), -65536, jnp.int32)

    def fire(u, buf):
        for h in range(CONN):
            pltpu.async_copy(xt_hbm.at[idx_v.at[h, pl.ds(u * UNIT, UNIT)]],
                             rows_v.at[buf, pl.ds(h * UNIT, UNIT)],
                             sems_g[buf])

    def drain_g(buf):
        for h in range(CONN):
            pltpu.make_async_copy(xt_hbm.at[idx_v.at[0, pl.ds(0, UNIT)]],
                                  rows_v.at[buf, pl.ds(h * UNIT, UNIT)],
                                  sems_g[buf]).wait()

    def compute(u, buf):
        def col(cc, carry):
            vv = plsc.load_gather(val_v, [kiota,
                                          jnp.full((16,), u * UNIT + cc,
                                                   jnp.int32)])
            accs = [jnp.zeros((16,), jnp.float32) for _ in range(4)]
            for k in range(CONN):
                s = vv[k]
                for half in range(2):
                    raw = rows_v[buf, k * UNIT + cc, pl.ds(half * 32, 32)]
                    bits = plsc.bitcast(raw, jnp.int32)
                    even = plsc.bitcast(lax.shift_left(bits, 16), jnp.float32)
                    odd = plsc.bitcast(bits & himask, jnp.float32)
                    accs[half * 2] = accs[half * 2] + s * even
                    accs[half * 2 + 1] = accs[half * 2 + 1] + s * odd
            ccv = jnp.full((16,), cc, jnp.int32)
            for j in range(4):
                boff = iota2 + ((j // 2) * 32 + (j % 2))
                plsc.store_scatter(out_v.at[buf], [ccv, boff], accs[j])
            return carry

        lax.fori_loop(0, UNIT, col, 0, unroll=2)

    def write_out(u, buf):
        pltpu.async_copy(out_v.at[buf],
                         out_hbm.at[pl.ds(row_base + u * UNIT, UNIT)],
                         sems_o[buf])

    def wait_out(buf):
        pltpu.make_async_copy(out_v.at[buf],
                              out_hbm.at[pl.ds(row_base, UNIT)],
                              sems_o[buf]).wait()

    fire(0, 0)
    fire(1, 1)
    drain_g(0)
    compute(0, 0)
    fire(2, 0)
    write_out(0, 0)
    drain_g(1)
    compute(1, 1)
    fire(3, 1)
    write_out(1, 1)

    def pair(t, carry):
        u0 = t * 2
        drain_g(0)
        wait_out(0)
        compute(u0, 0)
        fire(u0 + 2, 0)
        write_out(u0, 0)
        drain_g(1)
        wait_out(1)
        compute(u0 + 1, 1)
        fire(u0 + 3, 1)
        write_out(u0 + 1, 1)
        return carry

    lax.fori_loop(1, NUNIT // 2 - 1, pair, 0)

    drain_g(0)
    wait_out(0)
    compute(NUNIT - 2, 0)
    write_out(NUNIT - 2, 0)
    drain_g(1)
    wait_out(1)
    compute(NUNIT - 1, 1)
    write_out(NUNIT - 1, 1)
    wait_out(0)
    wait_out(1)


def kernel(x, indices, values):
    xt = x.astype(jnp.bfloat16).T
    outt = _spmm_sc(xt, indices[1], values)
    return outt.T


# speedup vs baseline: 3.5269x; 1.0282x over previous
import functools

import jax
import jax.numpy as jnp
from jax import lax
from jax.experimental import pallas as pl
from jax.experimental.pallas import tpu as pltpu
from jax.experimental.pallas import tpu_sc as plsc

IN_SIZE = 65536
OUT_SIZE = 65536
CONN = 16
BATCH = 64

NC = 2
NS = 16
NW = NC * NS
ROWS_PER_W = OUT_SIZE // NW
UNIT = 16
NUNIT = ROWS_PER_W // UNIT

_mesh = plsc.VectorSubcoreMesh(core_axis_name="c", subcore_axis_name="s")


@functools.partial(
    pl.kernel,
    mesh=_mesh,
    out_type=jax.ShapeDtypeStruct((OUT_SIZE, BATCH), jnp.float32),
    scratch_types=[
        pltpu.VMEM((CONN, ROWS_PER_W), jnp.int32),
        pltpu.VMEM((CONN, ROWS_PER_W), jnp.float32),
        pltpu.VMEM((2, CONN * UNIT, BATCH), jnp.float32),
        pltpu.VMEM((2, UNIT, BATCH), jnp.float32),
        pltpu.SemaphoreType.DMA,
        pltpu.SemaphoreType.DMA,
        pltpu.SemaphoreType.DMA,
        pltpu.SemaphoreType.DMA,
    ],
    compiler_params=pltpu.CompilerParams(use_tc_tiling_on_sc=False,
                                         needs_layout_passes=False),
)
def _spmm_sc(xt_hbm, idx_hbm, val_hbm, out_hbm, idx_v, val_v, rows_v, out_v,
             sem_g0, sem_g1, sem_o0, sem_o1):
    wid = lax.axis_index("s") * NC + lax.axis_index("c")
    row_base = wid * ROWS_PER_W
    sems_g = (sem_g0, sem_g1)
    sems_o = (sem_o0, sem_o1)

    for k in range(CONN):
        pltpu.async_copy(idx_hbm.at[pl.ds(k * OUT_SIZE + row_base, ROWS_PER_W)],
                         idx_v.at[k], sem_g0)
        pltpu.async_copy(val_hbm.at[pl.ds(k * OUT_SIZE + row_base, ROWS_PER_W)],
                         val_v.at[k], sem_g1)
    for k in range(CONN):
        pltpu.make_async_copy(idx_hbm.at[pl.ds(row_base, ROWS_PER_W)],
                              idx_v.at[k], sem_g0).wait()
        pltpu.make_async_copy(val_hbm.at[pl.ds(row_base, ROWS_PER_W)],
                              val_v.at[k], sem_g1).wait()

    kiota = lax.iota(jnp.int32, 16)

    def fire(u, buf):
        for h in range(CONN):
            pltpu.async_copy(xt_hbm.at[idx_v.at[h, pl.ds(u * UNIT, UNIT)]],
                             rows_v.at[buf, pl.ds(h * UNIT, UNIT)],
                             sems_g[buf])

    def drain_g(buf):
        for h in range(CONN):
            pltpu.make_async_copy(xt_hbm.at[idx_v.at[0, pl.ds(0, UNIT)]],
                                  rows_v.at[buf, pl.ds(h * UNIT, UNIT)],
                                  sems_g[buf]).wait()

    def compute(u, buf):
        def col(cc, carry):
            vv = plsc.load_gather(val_v, [kiota,
                                          jnp.full((16,), u * UNIT + cc,
                                                   jnp.int32)])
            accs = [jnp.zeros((16,), jnp.float32) for _ in range(BATCH // 16)]
            for k in range(CONN):
                s = vv[k]
                for j in range(BATCH // 16):
                    accs[j] = accs[j] + s * rows_v[buf, k * UNIT + cc,
                                                   pl.ds(j * 16, 16)]
            for j in range(BATCH // 16):
                out_v[buf, cc, pl.ds(j * 16, 16)] = accs[j]
            return carry

        lax.fori_loop(0, UNIT, col, 0)

    def write_out(u, buf):
        pltpu.async_copy(out_v.at[buf],
                         out_hbm.at[pl.ds(row_base + u * UNIT, UNIT)],
                         sems_o[buf])

    def wait_out(buf):
        pltpu.make_async_copy(out_v.at[buf],
                              out_hbm.at[pl.ds(row_base, UNIT)],
                              sems_o[buf]).wait()

    fire(0, 0)
    fire(1, 1)
    drain_g(0)
    compute(0, 0)
    fire(2, 0)
    write_out(0, 0)
    drain_g(1)
    compute(1, 1)
    fire(3, 1)
    write_out(1, 1)

    def pair(t, carry):
        u0 = t * 2
        drain_g(0)
        wait_out(0)
        compute(u0, 0)
        fire(u0 + 2, 0)
        write_out(u0, 0)
        drain_g(1)
        wait_out(1)
        compute(u0 + 1, 1)
        fire(u0 + 3, 1)
        write_out(u0 + 1, 1)
        return carry

    lax.fori_loop(1, NUNIT // 2 - 1, pair, 0)

    drain_g(0)
    wait_out(0)
    compute(NUNIT - 2, 0)
    write_out(NUNIT - 2, 0)
    drain_g(1)
    wait_out(1)
    compute(NUNIT - 1, 1)
    write_out(NUNIT - 1, 1)
    wait_out(0)
    wait_out(1)


def kernel(x, indices, values):
    xt = x.T
    outt = _spmm_sc(xt, indices[1], values)
    return outt.T
